# baseline (speedup 1.0000x reference)
import functools
import jax
import jax.numpy as jnp
from jax import lax
from jax.experimental import pallas as pl
from jax.experimental.pallas import tpu as pltpu

N = 10000
E = 160000
W = 32
KW2 = 64

NPAD = 10240
EPAD = 163840

_INTERP = False


def _node_enc_body(x_ref, w1_ref, b1_ref, w2_ref, b2_ref, g_ref, b_ref,
                   out_ref):
    h = jnp.maximum(
        jnp.dot(x_ref[...], w1_ref[...], preferred_element_type=jnp.float32)
        + b1_ref[...], 0.0)
    h = jnp.dot(h, w2_ref[...], preferred_element_type=jnp.float32) + b2_ref[...]
    mu = jnp.mean(h, axis=-1, keepdims=True)
    var = jnp.mean((h - mu) * (h - mu), axis=-1, keepdims=True)
    out_ref[...] = (h - mu) * jax.lax.rsqrt(var + 1e-5) * g_ref[...] + b_ref[...]


def _node_enc(x, ne):
    nb = 2048
    grid = NPAD // nb
    full = lambda s: pl.BlockSpec(s, lambda i: (0, 0))
    return pl.pallas_call(
        _node_enc_body,
        grid=(grid,),
        in_specs=[
            pl.BlockSpec((nb, 128), lambda i: (i, 0)),
            full((128, W)), full((1, W)), full((W, W)), full((1, W)),
            full((1, W)), full((1, W)),
        ],
        out_specs=pl.BlockSpec((nb, W), lambda i: (i, 0)),
        out_shape=jax.ShapeDtypeStruct((NPAD, W), jnp.float32),
        interpret=_INTERP,
    )(x, ne["l1"]["w"], ne["l1"]["b"].reshape(1, W),
      ne["l2"]["w"], ne["l2"]["b"].reshape(1, W),
      ne["ln_g"].reshape(1, W), ne["ln_b"].reshape(1, W))


def _edge_enc_body(eat_ref, w1_ref, b1_ref, w2_ref, b2_ref, g_ref, b_ref,
                   k11_ref, k11b_ref, k12_ref, k12b_ref,
                   k21_ref, k21b_ref, k22_ref, k22b_ref,
                   kf0_ref, kf1_ref):
    h = jnp.maximum(
        jnp.dot(w1_ref[...], eat_ref[...], preferred_element_type=jnp.float32)
        + b1_ref[...], 0.0)
    h = jnp.dot(w2_ref[...], h, preferred_element_type=jnp.float32) + b2_ref[...]
    mu = jnp.mean(h, axis=0, keepdims=True)
    var = jnp.mean((h - mu) * (h - mu), axis=0, keepdims=True)
    ea = (h - mu) * jax.lax.rsqrt(var + 1e-5) * g_ref[...] + b_ref[...]
    kf0 = jnp.maximum(
        jnp.dot(k11_ref[...], ea, preferred_element_type=jnp.float32)
        + k11b_ref[...], 0.0)
    kf0_ref[...] = jnp.maximum(
        jnp.dot(k12_ref[...], kf0, preferred_element_type=jnp.float32)
        + k12b_ref[...], 0.0)
    kf1 = jnp.maximum(
        jnp.dot(k21_ref[...], ea, preferred_element_type=jnp.float32)
        + k21b_ref[...], 0.0)
    kf1_ref[...] = jnp.maximum(
        jnp.dot(k22_ref[...], kf1, preferred_element_type=jnp.float32)
        + k22b_ref[...], 0.0)


def _edge_enc(eat, ee, proc):
    eb = 4096
    grid = EPAD // eb
    full = lambda s: pl.BlockSpec(s, lambda i: (0, 0))
    out_spec = pl.BlockSpec((KW2, eb), lambda i: (0, i))
    tw = lambda p: p["w"].T
    tb = lambda p, n: p["b"].reshape(n, 1)
    return pl.pallas_call(
        _edge_enc_body,
        grid=(grid,),
        in_specs=[
            pl.BlockSpec((4, eb), lambda i: (0, i)),
            full((W, 4)), full((W, 1)), full((W, W)), full((W, 1)),
            full((W, 1)), full((W, 1)),
            full((KW2, W)), full((KW2, 1)), full((KW2, KW2)), full((KW2, 1)),
            full((KW2, W)), full((KW2, 1)), full((KW2, KW2)), full((KW2, 1)),
        ],
        out_specs=[out_spec, out_spec],
        out_shape=[jax.ShapeDtypeStruct((KW2, EPAD), jnp.float32)] * 2,
        interpret=_INTERP,
    )(eat, tw(ee["l1"]), tb(ee["l1"], W), tw(ee["l2"]), tb(ee["l2"], W),
      ee["ln_g"].reshape(W, 1), ee["ln_b"].reshape(W, 1),
      tw(proc[0]["k1"]), tb(proc[0]["k1"], KW2),
      tw(proc[0]["k2"]), tb(proc[0]["k2"], KW2),
      tw(proc[1]["k1"]), tb(proc[1]["k1"], KW2),
      tw(proc[1]["k2"]), tb(proc[1]["k2"], KW2))


def _msg_body(fuse, kft_ref, hsa_ref, hsb_ref, w3t_ref, b3t_ref, out_ref):
    eb = kft_ref.shape[1]
    if fuse:
        hs = jnp.maximum(hsa_ref[...] + hsb_ref[...], 0.0)
    else:
        hs = hsa_ref[...]
    hst = hs.T
    kt = jnp.dot(w3t_ref[...], kft_ref[...],
                 preferred_element_type=jnp.float32) + b3t_ref[...]
    msgt = jnp.sum(kt.reshape(W, W, eb) * hst[None], axis=1)
    out_ref[...] = msgt.T


def _msg(kft, hsa, hsb, k3p, fuse):
    eb = 2048
    grid = EPAD // eb
    full = lambda s: pl.BlockSpec(s, lambda i: (0, 0))
    return pl.pallas_call(
        functools.partial(_msg_body, fuse),
        grid=(grid,),
        in_specs=[
            pl.BlockSpec((KW2, eb), lambda i: (0, i)),
            pl.BlockSpec((eb, W), lambda i: (i, 0)),
            pl.BlockSpec((eb, W), lambda i: (i, 0)),
            full((W * W, KW2)), full((W * W, 1)),
        ],
        out_specs=pl.BlockSpec((eb, W), lambda i: (i, 0)),
        out_shape=jax.ShapeDtypeStruct((EPAD, W), jnp.float32),
        interpret=_INTERP,
    )(kft, hsa, hsb, k3p["w"].T, k3p["b"].reshape(W * W, 1))


def _dec_body(q0_ref, q1_ref, w1_ref, b1_ref, w2_ref, b2_ref, out_ref):
    h = jnp.maximum(q0_ref[...] + q1_ref[...], 0.0)
    h = jnp.maximum(
        jnp.dot(h, w1_ref[...], preferred_element_type=jnp.float32)
        + b1_ref[...], 0.0)
    out_ref[...] = (jnp.dot(h, w2_ref[...], preferred_element_type=jnp.float32)
                    + b2_ref[...])


def _dec(q0, q1, dec):
    nb = 2048
    grid = NPAD // nb
    full = lambda s: pl.BlockSpec(s, lambda i: (0, 0))
    return pl.pallas_call(
        _dec_body,
        grid=(grid,),
        in_specs=[
            pl.BlockSpec((nb, W), lambda i: (i, 0)),
            pl.BlockSpec((nb, W), lambda i: (i, 0)),
            full((W, W)), full((1, W)), full((W, 8)), full((1, 8)),
        ],
        out_specs=pl.BlockSpec((nb, 8), lambda i: (i, 0)),
        out_shape=jax.ShapeDtypeStruct((NPAD, 8), jnp.float32),
        interpret=_INTERP,
    )(q0, q1, dec["l1"]["w"], dec["l1"]["b"].reshape(1, W),
      jnp.pad(dec["l2"]["w"], ((0, 0), (0, 5))),
      jnp.pad(dec["l2"]["b"], (0, 5)).reshape(1, 8))


from jax.experimental.pallas import tpu_sc as plsc

@functools.lru_cache(maxsize=None)
def _mesh():
    return plsc.VectorSubcoreMesh(core_axis_name="c", subcore_axis_name="s")


NW = 32
EC = 128
ROWS = EPAD // EC
RPW = ROWS // NW
RPC = ROWS // 2
RPT = RPC // 16
NPW = NPAD // NW
NPT = NPAD // 16


_NB = 4
_NBG = 4


def _gather_body(nt, *refs):
    tabs = refs[:nt]
    src_hbm = refs[nt]
    outs = refs[nt + 1:2 * nt + 1]
    idx_v = refs[2 * nt + 1]
    bufs = refs[2 * nt + 2:2 * nt + 2 + nt * _NBG]
    gsem = refs[2 * nt + 2 + nt * _NBG:2 * nt + 2 + 2 * nt * _NBG]
    wsem = refs[2 * nt + 2 + 2 * nt * _NBG:]
    c = lax.axis_index("c")
    s = lax.axis_index("s")
    w = s * 2 + c
    pltpu.sync_copy(src_hbm.at[pl.ds(w * RPW, RPW)], idx_v)
    ebase = w * RPW * EC
    gd = {}
    wd = {}
    for j in range(RPW + _NBG - 1):
        b = j % _NBG
        if j < RPW:
            for t in range(nt):
                k = t * _NBG + b
                if (j - _NBG, t) in wd:
                    wd.pop((j - _NBG, t)).wait()
                gd[(j, t)] = pltpu.async_copy(
                    tabs[t].at[idx_v.at[j]], bufs[k], gsem[k])
        jj = j - (_NBG - 1)
        if jj >= 0:
            bb = jj % _NBG
            for t in range(nt):
                k = t * _NBG + bb
                gd.pop((jj, t)).wait()
                wd[(jj, t)] = pltpu.async_copy(
                    bufs[k], outs[t].at[pl.ds(ebase + jj * EC, EC)], wsem[k])
    for key in sorted(wd):
        wd[key].wait()


def _sc_gather(tabs, src2d):
    nt = len(tabs)
    f = pl.kernel(
        functools.partial(_gather_body, nt),
        out_type=[jax.ShapeDtypeStruct((EPAD, W), jnp.float32)] * nt,
        mesh=_mesh(),
        compiler_params=pltpu.CompilerParams(use_tc_tiling_on_sc=False),
        scratch_types=(
            [pltpu.VMEM((RPW, EC), jnp.int32)]
            + [pltpu.VMEM((EC, W), jnp.float32)] * (nt * _NBG)
            + [pltpu.SemaphoreType.DMA] * (2 * nt * _NBG)
        ),
    )
    res = f(*tabs, src2d)
    return res if isinstance(res, (list, tuple)) else (res,)


def _scatter_body(first, msg_hbm, dst_hbm, ha_hbm, hb_hbm, zero_hbm,
                  q0_hbm, q1_hbm, idx_v, idc_v, ones_v, m0, m1, m2, m3,
                  acc_v, iv_v, ivr_v, ha_v, hb_v, acc_sh, cnt_sh,
                  l0, l1, l2, l3, s0, s1, s2, s3, csem):
    msg_v = (m0, m1, m2, m3)
    lsem = (l0, l1, l2, l3)
    ssem = (s0, s1, s2, s3)
    c = lax.axis_index("c")
    s = lax.axis_index("s")
    nb = s * NPT
    def _zero(i, _):
        iv_v[pl.ds(i * 16, 16)] = jnp.zeros((16,), jnp.float32)
        return 0
    lax.fori_loop(0, NPT // 32, _zero, 0)
    def _ones(i, _):
        ones_v[pl.ds(i * 16, 16)] = jnp.ones((16,), jnp.float32)
        return 0
    lax.fori_loop(0, EC // 16, _ones, 0, unroll=True)
    pltpu.sync_copy(zero_hbm.at[pl.ds(nb, NPT)], acc_sh.at[pl.ds(nb, NPT)])
    pltpu.sync_copy(iv_v, cnt_sh.at[pl.ds(nb, NPT // 2)])
    pltpu.sync_copy(iv_v, cnt_sh.at[pl.ds(nb + NPT // 2, NPT // 2)])
    plsc.subcore_barrier()
    pltpu.sync_copy(dst_hbm.at[pl.ds(s * 2 * RPT, 2 * RPT)], idc_v)
    cd = []
    for j in range(2 * RPT):
        cd.append(pltpu.async_copy(ones_v, cnt_sh.at[idc_v.at[j]], csem,
                                   add=True))
        if len(cd) >= 8:
            cd.pop(0).wait()
    base = c * RPC + s * RPT
    pltpu.sync_copy(dst_hbm.at[pl.ds(base, RPT)], idx_v)
    ld = {}
    sc = {}
    for j in range(RPT + _NB - 1):
        b = j % _NB
        if j < RPT:
            if (j - _NB) in sc:
                sc.pop(j - _NB).wait()
            ld[j] = pltpu.async_copy(
                msg_hbm.at[pl.ds((base + j) * EC, EC)], msg_v[b], lsem[b])
        jj = j - (_NB - 1)
        if jj >= 0:
            ld.pop(jj).wait()
            sc[jj] = pltpu.async_copy(
                msg_v[jj % _NB], acc_sh.at[idx_v.at[jj]], ssem[jj % _NB],
                add=True)
    for key in sorted(sc):
        sc[key].wait()
    for d in cd:
        d.wait()
    plsc.subcore_barrier()
    NH = NPT // 2
    for half in range(2):
        hb0 = nb + half * NH
        pltpu.sync_copy(acc_sh.at[pl.ds(hb0, NH)], acc_v)
        pltpu.sync_copy(cnt_sh.at[pl.ds(hb0, NH)], iv_v)
        def _inv(g, _):
            sl = pl.ds(g * 16, 16)
            v = 1.0 / jnp.maximum(iv_v[sl], 1.0)
            rows = g * 16 + lax.iota(jnp.int32, 16)
            for col in range(W):
                plsc.store_scatter(
                    ivr_v, [rows, jnp.full((16,), col, jnp.int32)], v)
            return 0
        lax.fori_loop(0, NH // 16, _inv, 0)

        @pl.when(c == 0)
        def _():
            def _mul(i, _):
                r = i // 2
                col = (i % 2) * 16
                acc_v[r, pl.ds(col, 16)] = (acc_v[r, pl.ds(col, 16)]
                                            * ivr_v[r, pl.ds(col, 16)])
                return 0
            lax.fori_loop(0, NH * 2, _mul, 0)
            pltpu.sync_copy(acc_v, q0_hbm.at[pl.ds(hb0, NH)])

        @pl.when(c == 1)
        def _():
            pltpu.sync_copy(ha_hbm.at[pl.ds(hb0, NH)], ha_v)
            if not first:
                pltpu.sync_copy(hb_hbm.at[pl.ds(hb0, NH)], hb_v)
            def _mad(i, _):
                r = i // 2
                col = (i % 2) * 16
                h = ha_v[r, pl.ds(col, 16)]
                if not first:
                    h = jnp.maximum(h + hb_v[r, pl.ds(col, 16)], 0.0)
                acc_v[r, pl.ds(col, 16)] = (acc_v[r, pl.ds(col, 16)]
                                            * ivr_v[r, pl.ds(col, 16)] + h)
                return 0
            lax.fori_loop(0, NH * 2, _mad, 0)
            pltpu.sync_copy(acc_v, q1_hbm.at[pl.ds(hb0, NH)])


def _sc_scatter(msg, dst2d, ha, hb, zeros2d, first):
    f = pl.kernel(
        functools.partial(_scatter_body, first),
        out_type=[jax.ShapeDtypeStruct((NPAD, W), jnp.float32)] * 2,
        mesh=_mesh(),
        compiler_params=pltpu.CompilerParams(use_tc_tiling_on_sc=False,
                                             needs_layout_passes=False),
        scratch_types=(
            [pltpu.VMEM((RPT, EC), jnp.int32)]
            + [pltpu.VMEM((2 * RPT, EC), jnp.int32)]
            + [pltpu.VMEM((EC,), jnp.float32)]
            + [pltpu.VMEM((EC, W), jnp.float32)] * _NB
            + [pltpu.VMEM((NPT // 2, W), jnp.float32)]
            + [pltpu.VMEM((NPT // 2,), jnp.float32)]
            + [pltpu.VMEM((NPT // 2, W), jnp.float32)] * 3
            + [pltpu.VMEM_SHARED((NPAD, W), jnp.float32)]
            + [pltpu.VMEM_SHARED((NPAD,), jnp.float32)]
            + [pltpu.SemaphoreType.DMA] * (2 * _NB + 1)
        ),
    )
    return f(msg, dst2d, ha, hb, zeros2d)


def kernel(x, edge_index, edge_attr, params):
    xp = jnp.pad(x, ((0, NPAD - N), (0, 0)))
    eat = jnp.pad(edge_attr, ((0, EPAD - E), (0, 0))).T
    src2d = jnp.pad(edge_index[0], (0, EPAD - E)).reshape(ROWS, EC)
    dst2d = jnp.pad(edge_index[1], (0, EPAD - E),
                    constant_values=NPAD - 1).reshape(ROWS, EC)
    zeros2d = jnp.zeros((NPAD, W), jnp.float32)

    h0 = _node_enc(xp, params["node_enc"])
    kf0, kf1 = _edge_enc(eat, params["edge_enc"], params["proc"])

    hs0, = _sc_gather([h0], src2d)
    msg0 = _msg(kf0, hs0, hs0, params["proc"][0]["k3"], fuse=False)
    q0a, q1a = _sc_scatter(msg0, dst2d, h0, h0, zeros2d, first=True)

    hs1a, hs1b = _sc_gather([q0a, q1a], src2d)
    msg1 = _msg(kf1, hs1a, hs1b, params["proc"][1]["k3"], fuse=True)
    q0b, q1b = _sc_scatter(msg1, dst2d, q0a, q1a, zeros2d, first=False)

    out = _dec(q0b, q1b, params["dec"])
    return out[:N, :3]


# speedup vs baseline: 1.0256x; 1.0256x over previous
import functools
import jax
import jax.numpy as jnp
from jax import lax
from jax.experimental import pallas as pl
from jax.experimental.pallas import tpu as pltpu

N = 10000
E = 160000
W = 32
KW2 = 64

NPAD = 10240
EPAD = 163840

_INTERP = False


def _node_enc_body(x_ref, c_ref, w1_ref, b1_ref, w2_ref, b2_ref, g_ref, b_ref,
                   out_ref, inv_ref):
    h = jnp.maximum(
        jnp.dot(x_ref[...], w1_ref[...], preferred_element_type=jnp.float32)
        + b1_ref[...], 0.0)
    h = jnp.dot(h, w2_ref[...], preferred_element_type=jnp.float32) + b2_ref[...]
    mu = jnp.mean(h, axis=-1, keepdims=True)
    var = jnp.mean((h - mu) * (h - mu), axis=-1, keepdims=True)
    out_ref[...] = (h - mu) * jax.lax.rsqrt(var + 1e-5) * g_ref[...] + b_ref[...]
    cnt = jnp.maximum(c_ref[...][:, 0:1] + c_ref[...][:, 1:2], 1.0)
    inv_ref[...] = jnp.broadcast_to(1.0 / cnt, inv_ref.shape)


def _node_enc(x, cT, ne):
    nb = 2048
    grid = NPAD // nb
    full = lambda s: pl.BlockSpec(s, lambda i: (0, 0))
    return pl.pallas_call(
        _node_enc_body,
        grid=(grid,),
        in_specs=[
            pl.BlockSpec((nb, 128), lambda i: (i, 0)),
            pl.BlockSpec((nb, 2), lambda i: (i, 0)),
            full((128, W)), full((1, W)), full((W, W)), full((1, W)),
            full((1, W)), full((1, W)),
        ],
        out_specs=[pl.BlockSpec((nb, W), lambda i: (i, 0))] * 2,
        out_shape=[jax.ShapeDtypeStruct((NPAD, W), jnp.float32)] * 2,
        interpret=_INTERP,
    )(x, cT, ne["l1"]["w"], ne["l1"]["b"].reshape(1, W),
      ne["l2"]["w"], ne["l2"]["b"].reshape(1, W),
      ne["ln_g"].reshape(1, W), ne["ln_b"].reshape(1, W))


def _msg_sub(kft, hst, w3t_ref, b3t_ref):
    eb = hst.shape[1]
    kt = jnp.dot(w3t_ref[...], kft,
                 preferred_element_type=jnp.float32) + b3t_ref[...]
    return jnp.sum(kt.reshape(W, W, eb) * hst[None], axis=1).T


def _msg0e_body(eat_ref, hs_ref, w1_ref, b1_ref, w2_ref, b2_ref, g_ref, b_ref,
                k11_ref, k11b_ref, k12_ref, k12b_ref,
                k21_ref, k21b_ref, k22_ref, k22b_ref,
                w3t_ref, b3t_ref, msg_ref, kf1_ref):
    h = jnp.maximum(
        jnp.dot(w1_ref[...], eat_ref[...], preferred_element_type=jnp.float32)
        + b1_ref[...], 0.0)
    h = jnp.dot(w2_ref[...], h, preferred_element_type=jnp.float32) + b2_ref[...]
    mu = jnp.mean(h, axis=0, keepdims=True)
    var = jnp.mean((h - mu) * (h - mu), axis=0, keepdims=True)
    ea = (h - mu) * jax.lax.rsqrt(var + 1e-5) * g_ref[...] + b_ref[...]
    kf0 = jnp.maximum(
        jnp.dot(k11_ref[...], ea, preferred_element_type=jnp.float32)
        + k11b_ref[...], 0.0)
    kf0 = jnp.maximum(
        jnp.dot(k12_ref[...], kf0, preferred_element_type=jnp.float32)
        + k12b_ref[...], 0.0)
    kf1 = jnp.maximum(
        jnp.dot(k21_ref[...], ea, preferred_element_type=jnp.float32)
        + k21b_ref[...], 0.0)
    kf1_ref[...] = jnp.maximum(
        jnp.dot(k22_ref[...], kf1, preferred_element_type=jnp.float32)
        + k22b_ref[...], 0.0)
    msg_ref[...] = _msg_sub(kf0, hs_ref[...].T, w3t_ref, b3t_ref)


def _msg0e(eat, hs, ee, proc):
    eb = 2048
    grid = EPAD // eb
    full = lambda s: pl.BlockSpec(s, lambda i: (0, 0))
    tw = lambda p: p["w"].T
    tb = lambda p, n: p["b"].reshape(n, 1)
    return pl.pallas_call(
        _msg0e_body,
        grid=(grid,),
        in_specs=[
            pl.BlockSpec((4, eb), lambda i: (0, i)),
            pl.BlockSpec((eb, W), lambda i: (i, 0)),
            full((W, 4)), full((W, 1)), full((W, W)), full((W, 1)),
            full((W, 1)), full((W, 1)),
            full((KW2, W)), full((KW2, 1)), full((KW2, KW2)), full((KW2, 1)),
            full((KW2, W)), full((KW2, 1)), full((KW2, KW2)), full((KW2, 1)),
            full((W * W, KW2)), full((W * W, 1)),
        ],
        out_specs=[pl.BlockSpec((eb, W), lambda i: (i, 0)),
                   pl.BlockSpec((KW2, eb), lambda i: (0, i))],
        out_shape=[jax.ShapeDtypeStruct((EPAD, W), jnp.float32),
                   jax.ShapeDtypeStruct((KW2, EPAD), jnp.float32)],
        interpret=_INTERP,
    )(eat, hs, tw(ee["l1"]), tb(ee["l1"], W), tw(ee["l2"]), tb(ee["l2"], W),
      ee["ln_g"].reshape(W, 1), ee["ln_b"].reshape(W, 1),
      tw(proc[0]["k1"]), tb(proc[0]["k1"], KW2),
      tw(proc[0]["k2"]), tb(proc[0]["k2"], KW2),
      tw(proc[1]["k1"]), tb(proc[1]["k1"], KW2),
      tw(proc[1]["k2"]), tb(proc[1]["k2"], KW2),
      proc[0]["k3"]["w"].T, proc[0]["k3"]["b"].reshape(W * W, 1))


def _msg_body(fuse, kft_ref, hsa_ref, hsb_ref, w3t_ref, b3t_ref, out_ref):
    if fuse:
        hs = jnp.maximum(hsa_ref[...] + hsb_ref[...], 0.0)
    else:
        hs = hsa_ref[...]
    out_ref[...] = _msg_sub(kft_ref[...], hs.T, w3t_ref, b3t_ref)


def _msg(kft, hsa, hsb, k3p, fuse):
    eb = 2048
    grid = EPAD // eb
    full = lambda s: pl.BlockSpec(s, lambda i: (0, 0))
    return pl.pallas_call(
        functools.partial(_msg_body, fuse),
        grid=(grid,),
        in_specs=[
            pl.BlockSpec((KW2, eb), lambda i: (0, i)),
            pl.BlockSpec((eb, W), lambda i: (i, 0)),
            pl.BlockSpec((eb, W), lambda i: (i, 0)),
            full((W * W, KW2)), full((W * W, 1)),
        ],
        out_specs=pl.BlockSpec((eb, W), lambda i: (i, 0)),
        out_shape=jax.ShapeDtypeStruct((EPAD, W), jnp.float32),
        interpret=_INTERP,
    )(kft, hsa, hsb, k3p["w"].T, k3p["b"].reshape(W * W, 1))


def _dec_body(q0_ref, q1_ref, w1_ref, b1_ref, w2_ref, b2_ref, out_ref):
    h = jnp.maximum(q0_ref[...] + q1_ref[...], 0.0)
    h = jnp.maximum(
        jnp.dot(h, w1_ref[...], preferred_element_type=jnp.float32)
        + b1_ref[...], 0.0)
    out_ref[...] = (jnp.dot(h, w2_ref[...], preferred_element_type=jnp.float32)
                    + b2_ref[...])


def _dec(q0, q1, dec):
    nb = 2048
    grid = NPAD // nb
    full = lambda s: pl.BlockSpec(s, lambda i: (0, 0))
    return pl.pallas_call(
        _dec_body,
        grid=(grid,),
        in_specs=[
            pl.BlockSpec((nb, W), lambda i: (i, 0)),
            pl.BlockSpec((nb, W), lambda i: (i, 0)),
            full((W, W)), full((1, W)), full((W, 8)), full((1, 8)),
        ],
        out_specs=pl.BlockSpec((nb, 8), lambda i: (i, 0)),
        out_shape=jax.ShapeDtypeStruct((NPAD, 8), jnp.float32),
        interpret=_INTERP,
    )(q0, q1, dec["l1"]["w"], dec["l1"]["b"].reshape(1, W),
      jnp.pad(dec["l2"]["w"], ((0, 0), (0, 5))),
      jnp.pad(dec["l2"]["b"], (0, 5)).reshape(1, 8))


from jax.experimental.pallas import tpu_sc as plsc

@functools.lru_cache(maxsize=None)
def _mesh():
    return plsc.VectorSubcoreMesh(core_axis_name="c", subcore_axis_name="s")


NW = 32
EC = 128
ROWS = EPAD // EC
RPW = ROWS // NW
RPC = ROWS // 2
RPT = RPC // 16
NPW = NPAD // NW
NPT = NPAD // 16


_NB = 4
_NBG = 4


def _gather_body(nt, *refs):
    tabs = refs[:nt]
    src_hbm = refs[nt]
    outs = refs[nt + 1:2 * nt + 1]
    idx_v = refs[2 * nt + 1]
    bufs = refs[2 * nt + 2:2 * nt + 2 + nt * _NBG]
    gsem = refs[2 * nt + 2 + nt * _NBG:2 * nt + 2 + 2 * nt * _NBG]
    wsem = refs[2 * nt + 2 + 2 * nt * _NBG:]
    c = lax.axis_index("c")
    s = lax.axis_index("s")
    w = s * 2 + c
    pltpu.sync_copy(src_hbm.at[pl.ds(w * RPW, RPW)], idx_v)
    ebase = w * RPW * EC
    gd = {}
    wd = {}
    for j in range(RPW + _NBG - 1):
        b = j % _NBG
        if j < RPW:
            for t in range(nt):
                k = t * _NBG + b
                if (j - _NBG, t) in wd:
                    wd.pop((j - _NBG, t)).wait()
                gd[(j, t)] = pltpu.async_copy(
                    tabs[t].at[idx_v.at[j]], bufs[k], gsem[k])
        jj = j - (_NBG - 1)
        if jj >= 0:
            bb = jj % _NBG
            for t in range(nt):
                k = t * _NBG + bb
                gd.pop((jj, t)).wait()
                wd[(jj, t)] = pltpu.async_copy(
                    bufs[k], outs[t].at[pl.ds(ebase + jj * EC, EC)], wsem[k])
    for key in sorted(wd):
        wd[key].wait()


def _sc_gather(tabs, src2d):
    nt = len(tabs)
    f = pl.kernel(
        functools.partial(_gather_body, nt),
        out_type=[jax.ShapeDtypeStruct((EPAD, W), jnp.float32)] * nt,
        mesh=_mesh(),
        compiler_params=pltpu.CompilerParams(use_tc_tiling_on_sc=False),
        scratch_types=(
            [pltpu.VMEM((RPW, EC), jnp.int32)]
            + [pltpu.VMEM((EC, W), jnp.float32)] * (nt * _NBG)
            + [pltpu.SemaphoreType.DMA] * (2 * nt * _NBG)
        ),
    )
    res = f(*tabs, src2d)
    return res if isinstance(res, (list, tuple)) else (res,)


def _count_body(dst_hbm, zero_hbm, out_hbm, idx_v, ones_v, acc_sh, sem):
    c = lax.axis_index("c")
    s = lax.axis_index("s")
    def _ones(i, _):
        ones_v[pl.ds(i * 16, 16)] = jnp.ones((16,), jnp.float32)
        return 0
    lax.fori_loop(0, EC // 16, _ones, 0, unroll=True)
    pltpu.sync_copy(zero_hbm.at[pl.ds(s * NPT, NPT)],
                    acc_sh.at[pl.ds(s * NPT, NPT)])
    plsc.subcore_barrier()
    base = c * RPC + s * RPT
    pltpu.sync_copy(dst_hbm.at[pl.ds(base, RPT)], idx_v)
    cd = []
    for j in range(RPT):
        cd.append(pltpu.async_copy(ones_v, acc_sh.at[idx_v.at[j]], sem,
                                   add=True))
        if len(cd) >= 4:
            cd.pop(0).wait()
    for d in cd:
        d.wait()
    plsc.subcore_barrier()
    pltpu.sync_copy(acc_sh.at[pl.ds(s * NPT, NPT)],
                    out_hbm.at[c, pl.ds(s * NPT, NPT)])


def _sc_count(dst2d, zeros1d):
    f = pl.kernel(
        _count_body,
        out_type=jax.ShapeDtypeStruct((2, NPAD), jnp.float32),
        mesh=_mesh(),
        compiler_params=pltpu.CompilerParams(use_tc_tiling_on_sc=False),
        scratch_types=[
            pltpu.VMEM((RPT, EC), jnp.int32),
            pltpu.VMEM((EC,), jnp.float32),
            pltpu.VMEM_SHARED((NPAD,), jnp.float32),
            pltpu.SemaphoreType.DMA,
        ],
    )
    return f(dst2d, zeros1d)


def _scatter_body(first, msg_hbm, dst_hbm, ha_hbm, hb_hbm, inv_hbm, zero_hbm,
                  q0_hbm, q1_hbm, idx_v, m0, m1, m2, m3, acc_v, iv_v, ha_v,
                  hb_v, acc_sh, l0, l1, l2, l3, s0, s1, s2, s3):
    msg_v = (m0, m1, m2, m3)
    lsem = (l0, l1, l2, l3)
    ssem = (s0, s1, s2, s3)
    c = lax.axis_index("c")
    s = lax.axis_index("s")
    nb = s * NPT
    pltpu.sync_copy(zero_hbm.at[pl.ds(nb, NPT)], acc_sh.at[pl.ds(nb, NPT)])
    plsc.subcore_barrier()
    base = c * RPC + s * RPT
    pltpu.sync_copy(dst_hbm.at[pl.ds(base, RPT)], idx_v)
    ld = {}
    sc = {}
    for j in range(RPT + _NB - 1):
        b = j % _NB
        if j < RPT:
            if (j - _NB) in sc:
                sc.pop(j - _NB).wait()
            ld[j] = pltpu.async_copy(
                msg_hbm.at[pl.ds((base + j) * EC, EC)], msg_v[b], lsem[b])
        jj = j - (_NB - 1)
        if jj >= 0:
            ld.pop(jj).wait()
            sc[jj] = pltpu.async_copy(
                msg_v[jj % _NB], acc_sh.at[idx_v.at[jj]], ssem[jj % _NB],
                add=True)
    for key in sorted(sc):
        sc[key].wait()
    plsc.subcore_barrier()
    pltpu.sync_copy(acc_sh.at[pl.ds(nb, NPT)], acc_v)
    pltpu.sync_copy(inv_hbm.at[pl.ds(nb, NPT)], iv_v)

    @pl.when(c == 0)
    def _():
        def _mul(i, _):
            r = i // 2
            col = (i % 2) * 16
            acc_v[r, pl.ds(col, 16)] = (acc_v[r, pl.ds(col, 16)]
                                        * iv_v[r, pl.ds(col, 16)])
            return 0
        lax.fori_loop(0, NPT * 2, _mul, 0)
        pltpu.sync_copy(acc_v, q0_hbm.at[pl.ds(nb, NPT)])

    @pl.when(c == 1)
    def _():
        pltpu.sync_copy(ha_hbm.at[pl.ds(nb, NPT)], ha_v)
        if not first:
            pltpu.sync_copy(hb_hbm.at[pl.ds(nb, NPT)], hb_v)
        def _mad(i, _):
            r = i // 2
            col = (i % 2) * 16
            h = ha_v[r, pl.ds(col, 16)]
            if not first:
                h = jnp.maximum(h + hb_v[r, pl.ds(col, 16)], 0.0)
            acc_v[r, pl.ds(col, 16)] = (acc_v[r, pl.ds(col, 16)]
                                        * iv_v[r, pl.ds(col, 16)] + h)
            return 0
        lax.fori_loop(0, NPT * 2, _mad, 0)
        pltpu.sync_copy(acc_v, q1_hbm.at[pl.ds(nb, NPT)])


def _sc_scatter(msg, dst2d, ha, hb, invrep, zeros2d, first):
    f = pl.kernel(
        functools.partial(_scatter_body, first),
        out_type=[jax.ShapeDtypeStruct((NPAD, W), jnp.float32)] * 2,
        mesh=_mesh(),
        compiler_params=pltpu.CompilerParams(use_tc_tiling_on_sc=False),
        scratch_types=(
            [pltpu.VMEM((RPT, EC), jnp.int32)]
            + [pltpu.VMEM((EC, W), jnp.float32)] * _NB
            + [pltpu.VMEM((NPT, W), jnp.float32)] * 4
            + [pltpu.VMEM_SHARED((NPAD, W), jnp.float32)]
            + [pltpu.SemaphoreType.DMA] * (2 * _NB)
        ),
    )
    return f(msg, dst2d, ha, hb, invrep, zeros2d)


def kernel(x, edge_index, edge_attr, params):
    xp = jnp.pad(x, ((0, NPAD - N), (0, 0)))
    eat = jnp.pad(edge_attr, ((0, EPAD - E), (0, 0))).T
    src2d = jnp.pad(edge_index[0], (0, EPAD - E)).reshape(ROWS, EC)
    dst2d = jnp.pad(edge_index[1], (0, EPAD - E),
                    constant_values=NPAD - 1).reshape(ROWS, EC)
    zeros1d = jnp.zeros((NPAD,), jnp.float32)
    zeros2d = jnp.zeros((NPAD, W), jnp.float32)

    cnt = _sc_count(dst2d, zeros1d)
    h0, invrep = _node_enc(xp, cnt.T, params["node_enc"])

    hs0, = _sc_gather([h0], src2d)
    msg0, kf1 = _msg0e(eat, hs0, params["edge_enc"], params["proc"])
    q0a, q1a = _sc_scatter(msg0, dst2d, h0, h0, invrep, zeros2d, first=True)

    hs1a, hs1b = _sc_gather([q0a, q1a], src2d)
    msg1 = _msg(kf1, hs1a, hs1b, params["proc"][1]["k3"], fuse=True)
    q0b, q1b = _sc_scatter(msg1, dst2d, q0a, q1a, invrep, zeros2d, first=False)

    out = _dec(q0b, q1b, params["dec"])
    return out[:N, :3]


# speedup vs baseline: 1.0378x; 1.0119x over previous
import functools
import jax
import jax.numpy as jnp
from jax import lax
from jax.experimental import pallas as pl
from jax.experimental.pallas import tpu as pltpu

N = 10000
E = 160000
W = 32
KW2 = 64

NPAD = 10240
EPAD = 163840

_INTERP = False


def _node_enc_body(x_ref, c_ref, w1_ref, b1_ref, w2_ref, b2_ref, g_ref, b_ref,
                   out_ref, inv_ref):
    h = jnp.maximum(
        jnp.dot(x_ref[...], w1_ref[...], preferred_element_type=jnp.float32)
        + b1_ref[...], 0.0)
    h = jnp.dot(h, w2_ref[...], preferred_element_type=jnp.float32) + b2_ref[...]
    mu = jnp.mean(h, axis=-1, keepdims=True)
    var = jnp.mean((h - mu) * (h - mu), axis=-1, keepdims=True)
    out_ref[...] = (h - mu) * jax.lax.rsqrt(var + 1e-5) * g_ref[...] + b_ref[...]
    cnt = jnp.maximum(c_ref[...][:, 0:1] + c_ref[...][:, 1:2], 1.0)
    inv_ref[...] = jnp.broadcast_to(1.0 / cnt, inv_ref.shape)


def _node_enc(x, cT, ne):
    nb = 2048
    grid = NPAD // nb
    full = lambda s: pl.BlockSpec(s, lambda i: (0, 0))
    return pl.pallas_call(
        _node_enc_body,
        grid=(grid,),
        in_specs=[
            pl.BlockSpec((nb, 128), lambda i: (i, 0)),
            pl.BlockSpec((nb, 2), lambda i: (i, 0)),
            full((128, W)), full((1, W)), full((W, W)), full((1, W)),
            full((1, W)), full((1, W)),
        ],
        out_specs=[pl.BlockSpec((nb, W), lambda i: (i, 0))] * 2,
        out_shape=[jax.ShapeDtypeStruct((NPAD, W), jnp.float32)] * 2,
        interpret=_INTERP,
    )(x, cT, ne["l1"]["w"], ne["l1"]["b"].reshape(1, W),
      ne["l2"]["w"], ne["l2"]["b"].reshape(1, W),
      ne["ln_g"].reshape(1, W), ne["ln_b"].reshape(1, W))


def _msg_sub(kft, hst, w3t_ref, b3t_ref):
    eb = hst.shape[1]
    kt = jnp.dot(w3t_ref[...], kft,
                 preferred_element_type=jnp.float32) + b3t_ref[...]
    return jnp.sum(kt.reshape(W, W, eb) * hst[None], axis=1).T


def _edge_enc_body(eat_ref, w1_ref, b1_ref, w2_ref, b2_ref, g_ref, b_ref,
                   k11_ref, k11b_ref, k12_ref, k12b_ref,
                   k21_ref, k21b_ref, k22_ref, k22b_ref,
                   kf0_ref, kf1_ref):
    h = jnp.maximum(
        jnp.dot(w1_ref[...], eat_ref[...], preferred_element_type=jnp.float32)
        + b1_ref[...], 0.0)
    h = jnp.dot(w2_ref[...], h, preferred_element_type=jnp.float32) + b2_ref[...]
    mu = jnp.mean(h, axis=0, keepdims=True)
    var = jnp.mean((h - mu) * (h - mu), axis=0, keepdims=True)
    ea = (h - mu) * jax.lax.rsqrt(var + 1e-5) * g_ref[...] + b_ref[...]
    kf0 = jnp.maximum(
        jnp.dot(k11_ref[...], ea, preferred_element_type=jnp.float32)
        + k11b_ref[...], 0.0)
    kf0_ref[...] = jnp.maximum(
        jnp.dot(k12_ref[...], kf0, preferred_element_type=jnp.float32)
        + k12b_ref[...], 0.0)
    kf1 = jnp.maximum(
        jnp.dot(k21_ref[...], ea, preferred_element_type=jnp.float32)
        + k21b_ref[...], 0.0)
    kf1_ref[...] = jnp.maximum(
        jnp.dot(k22_ref[...], kf1, preferred_element_type=jnp.float32)
        + k22b_ref[...], 0.0)


def _edge_enc(eat, ee, proc):
    eb = 4096
    grid = EPAD // eb
    full = lambda s: pl.BlockSpec(s, lambda i: (0, 0))
    out_spec = pl.BlockSpec((KW2, eb), lambda i: (0, i))
    tw = lambda p: p["w"].T
    tb = lambda p, n: p["b"].reshape(n, 1)
    return pl.pallas_call(
        _edge_enc_body,
        grid=(grid,),
        in_specs=[
            pl.BlockSpec((4, eb), lambda i: (0, i)),
            full((W, 4)), full((W, 1)), full((W, W)), full((W, 1)),
            full((W, 1)), full((W, 1)),
            full((KW2, W)), full((KW2, 1)), full((KW2, KW2)), full((KW2, 1)),
            full((KW2, W)), full((KW2, 1)), full((KW2, KW2)), full((KW2, 1)),
        ],
        out_specs=[out_spec, out_spec],
        out_shape=[jax.ShapeDtypeStruct((KW2, EPAD), jnp.float32)] * 2,
        interpret=_INTERP,
    )(eat, tw(ee["l1"]), tb(ee["l1"], W), tw(ee["l2"]), tb(ee["l2"], W),
      ee["ln_g"].reshape(W, 1), ee["ln_b"].reshape(W, 1),
      tw(proc[0]["k1"]), tb(proc[0]["k1"], KW2),
      tw(proc[0]["k2"]), tb(proc[0]["k2"], KW2),
      tw(proc[1]["k1"]), tb(proc[1]["k1"], KW2),
      tw(proc[1]["k2"]), tb(proc[1]["k2"], KW2))


def _msg_body(fuse, kft_ref, hsa_ref, hsb_ref, w3t_ref, b3t_ref, out_ref):
    if fuse:
        hs = jnp.maximum(hsa_ref[...] + hsb_ref[...], 0.0)
    else:
        hs = hsa_ref[...]
    out_ref[...] = _msg_sub(kft_ref[...], hs.T, w3t_ref, b3t_ref)


def _msg(kft, hsa, hsb, k3p, fuse, half):
    eb = 2048
    grid = (EPAD // 2) // eb
    off = half * grid
    full = lambda s: pl.BlockSpec(s, lambda i: (0, 0))
    return pl.pallas_call(
        functools.partial(_msg_body, fuse),
        grid=(grid,),
        in_specs=[
            pl.BlockSpec((KW2, eb), lambda i: (0, off + i)),
            pl.BlockSpec((eb, W), lambda i: (i, 0)),
            pl.BlockSpec((eb, W), lambda i: (i, 0)),
            full((W * W, KW2)), full((W * W, 1)),
        ],
        out_specs=pl.BlockSpec((eb, W), lambda i: (i, 0)),
        out_shape=jax.ShapeDtypeStruct((EPAD // 2, W), jnp.float32),
        interpret=_INTERP,
    )(kft, hsa, hsb, k3p["w"].T, k3p["b"].reshape(W * W, 1))


def _dec_body(q0_ref, q1_ref, w1_ref, b1_ref, w2_ref, b2_ref, out_ref):
    h = jnp.maximum(q0_ref[...] + q1_ref[...], 0.0)
    h = jnp.maximum(
        jnp.dot(h, w1_ref[...], preferred_element_type=jnp.float32)
        + b1_ref[...], 0.0)
    out_ref[...] = (jnp.dot(h, w2_ref[...], preferred_element_type=jnp.float32)
                    + b2_ref[...])


def _dec(q0, q1, dec):
    nb = 2048
    grid = NPAD // nb
    full = lambda s: pl.BlockSpec(s, lambda i: (0, 0))
    return pl.pallas_call(
        _dec_body,
        grid=(grid,),
        in_specs=[
            pl.BlockSpec((nb, W), lambda i: (i, 0)),
            pl.BlockSpec((nb, W), lambda i: (i, 0)),
            full((W, W)), full((1, W)), full((W, 8)), full((1, 8)),
        ],
        out_specs=pl.BlockSpec((nb, 8), lambda i: (i, 0)),
        out_shape=jax.ShapeDtypeStruct((NPAD, 8), jnp.float32),
        interpret=_INTERP,
    )(q0, q1, dec["l1"]["w"], dec["l1"]["b"].reshape(1, W),
      jnp.pad(dec["l2"]["w"], ((0, 0), (0, 5))),
      jnp.pad(dec["l2"]["b"], (0, 5)).reshape(1, 8))


from jax.experimental.pallas import tpu_sc as plsc

@functools.lru_cache(maxsize=None)
def _mesh():
    return plsc.VectorSubcoreMesh(core_axis_name="c", subcore_axis_name="s")


NW = 32
EC = 128
ROWS = EPAD // EC
RPW = ROWS // NW
RPC = ROWS // 2
RPT = RPC // 16
NPW = NPAD // NW
NPT = NPAD // 16


_NB = 4
_NBG = 4


RPH = RPW // 2


def _gather_body(nt, half, *refs):
    tabs = refs[:nt]
    src_hbm = refs[nt]
    outs = refs[nt + 1:2 * nt + 1]
    idx_v = refs[2 * nt + 1]
    bufs = refs[2 * nt + 2:2 * nt + 2 + nt * _NBG]
    gsem = refs[2 * nt + 2 + nt * _NBG:2 * nt + 2 + 2 * nt * _NBG]
    wsem = refs[2 * nt + 2 + 2 * nt * _NBG:]
    c = lax.axis_index("c")
    s = lax.axis_index("s")
    w = s * 2 + c
    pltpu.sync_copy(src_hbm.at[pl.ds(half * RPC + w * RPH, RPH)], idx_v)
    ebase = w * RPH * EC
    gd = {}
    wd = {}
    for j in range(RPH + _NBG - 1):
        b = j % _NBG
        if j < RPH:
            for t in range(nt):
                k = t * _NBG + b
                if (j - _NBG, t) in wd:
                    wd.pop((j - _NBG, t)).wait()
                gd[(j, t)] = pltpu.async_copy(
                    tabs[t].at[idx_v.at[j]], bufs[k], gsem[k])
        jj = j - (_NBG - 1)
        if jj >= 0:
            bb = jj % _NBG
            for t in range(nt):
                k = t * _NBG + bb
                gd.pop((jj, t)).wait()
                wd[(jj, t)] = pltpu.async_copy(
                    bufs[k], outs[t].at[pl.ds(ebase + jj * EC, EC)], wsem[k])
    for key in sorted(wd):
        wd[key].wait()


def _sc_gather(tabs, src2d, half):
    nt = len(tabs)
    f = pl.kernel(
        functools.partial(_gather_body, nt, half),
        out_type=[jax.ShapeDtypeStruct((EPAD // 2, W), jnp.float32)] * nt,
        mesh=_mesh(),
        compiler_params=pltpu.CompilerParams(use_tc_tiling_on_sc=False),
        scratch_types=(
            [pltpu.VMEM((RPH, EC), jnp.int32)]
            + [pltpu.VMEM((EC, W), jnp.float32)] * (nt * _NBG)
            + [pltpu.SemaphoreType.DMA] * (2 * nt * _NBG)
        ),
    )
    res = f(*tabs, src2d)
    return res if isinstance(res, (list, tuple)) else (res,)


def _count_body(dst_hbm, zero_hbm, out_hbm, idx_v, ones_v, acc_sh, sem):
    c = lax.axis_index("c")
    s = lax.axis_index("s")
    def _ones(i, _):
        ones_v[pl.ds(i * 16, 16)] = jnp.ones((16,), jnp.float32)
        return 0
    lax.fori_loop(0, EC // 16, _ones, 0, unroll=True)
    pltpu.sync_copy(zero_hbm.at[pl.ds(s * NPT, NPT)],
                    acc_sh.at[pl.ds(s * NPT, NPT)])
    plsc.subcore_barrier()
    base = c * RPC + s * RPT
    pltpu.sync_copy(dst_hbm.at[pl.ds(base, RPT)], idx_v)
    cd = []
    for j in range(RPT):
        cd.append(pltpu.async_copy(ones_v, acc_sh.at[idx_v.at[j]], sem,
                                   add=True))
        if len(cd) >= 4:
            cd.pop(0).wait()
    for d in cd:
        d.wait()
    plsc.subcore_barrier()
    pltpu.sync_copy(acc_sh.at[pl.ds(s * NPT, NPT)],
                    out_hbm.at[c, pl.ds(s * NPT, NPT)])


def _sc_count(dst2d, zeros1d):
    f = pl.kernel(
        _count_body,
        out_type=jax.ShapeDtypeStruct((2, NPAD), jnp.float32),
        mesh=_mesh(),
        compiler_params=pltpu.CompilerParams(use_tc_tiling_on_sc=False),
        scratch_types=[
            pltpu.VMEM((RPT, EC), jnp.int32),
            pltpu.VMEM((EC,), jnp.float32),
            pltpu.VMEM_SHARED((NPAD,), jnp.float32),
            pltpu.SemaphoreType.DMA,
        ],
    )
    return f(dst2d, zeros1d)


def _scatter_body(first, msga_hbm, msgb_hbm, dst_hbm, ha_hbm, hb_hbm, inv_hbm,
                  zero_hbm, q0_hbm, q1_hbm, idx_v, m0, m1, m2, m3, acc_v,
                  iv_v, ha_v, hb_v, acc_sh, l0, l1, l2, l3, s0, s1, s2, s3):
    msg_v = (m0, m1, m2, m3)
    lsem = (l0, l1, l2, l3)
    ssem = (s0, s1, s2, s3)
    c = lax.axis_index("c")
    s = lax.axis_index("s")
    nb = s * NPT
    pltpu.sync_copy(zero_hbm.at[pl.ds(nb, NPT)], acc_sh.at[pl.ds(nb, NPT)])
    plsc.subcore_barrier()
    base = c * RPC + s * RPT
    pltpu.sync_copy(dst_hbm.at[pl.ds(base, RPT)], idx_v)

    def _accum(msg_hbm):
        ld = {}
        sc = {}
        for j in range(RPT + _NB - 1):
            b = j % _NB
            if j < RPT:
                if (j - _NB) in sc:
                    sc.pop(j - _NB).wait()
                ld[j] = pltpu.async_copy(
                    msg_hbm.at[pl.ds((s * RPT + j) * EC, EC)], msg_v[b],
                    lsem[b])
            jj = j - (_NB - 1)
            if jj >= 0:
                ld.pop(jj).wait()
                sc[jj] = pltpu.async_copy(
                    msg_v[jj % _NB], acc_sh.at[idx_v.at[jj]], ssem[jj % _NB],
                    add=True)
        for key in sorted(sc):
            sc[key].wait()

    @pl.when(c == 0)
    def _():
        _accum(msga_hbm)

    @pl.when(c == 1)
    def _():
        _accum(msgb_hbm)
    plsc.subcore_barrier()
    pltpu.sync_copy(acc_sh.at[pl.ds(nb, NPT)], acc_v)
    pltpu.sync_copy(inv_hbm.at[pl.ds(nb, NPT)], iv_v)

    @pl.when(c == 0)
    def _():
        def _mul(i, _):
            r = i // 2
            col = (i % 2) * 16
            acc_v[r, pl.ds(col, 16)] = (acc_v[r, pl.ds(col, 16)]
                                        * iv_v[r, pl.ds(col, 16)])
            return 0
        lax.fori_loop(0, NPT * 2, _mul, 0)
        pltpu.sync_copy(acc_v, q0_hbm.at[pl.ds(nb, NPT)])

    @pl.when(c == 1)
    def _():
        pltpu.sync_copy(ha_hbm.at[pl.ds(nb, NPT)], ha_v)
        if not first:
            pltpu.sync_copy(hb_hbm.at[pl.ds(nb, NPT)], hb_v)
        def _mad(i, _):
            r = i // 2
            col = (i % 2) * 16
            h = ha_v[r, pl.ds(col, 16)]
            if not first:
                h = jnp.maximum(h + hb_v[r, pl.ds(col, 16)], 0.0)
            acc_v[r, pl.ds(col, 16)] = (acc_v[r, pl.ds(col, 16)]
                                        * iv_v[r, pl.ds(col, 16)] + h)
            return 0
        lax.fori_loop(0, NPT * 2, _mad, 0)
        pltpu.sync_copy(acc_v, q1_hbm.at[pl.ds(nb, NPT)])


def _sc_scatter(msga, msgb, dst2d, ha, hb, invrep, zeros2d, first):
    f = pl.kernel(
        functools.partial(_scatter_body, first),
        out_type=[jax.ShapeDtypeStruct((NPAD, W), jnp.float32)] * 2,
        mesh=_mesh(),
        compiler_params=pltpu.CompilerParams(use_tc_tiling_on_sc=False),
        scratch_types=(
            [pltpu.VMEM((RPT, EC), jnp.int32)]
            + [pltpu.VMEM((EC, W), jnp.float32)] * _NB
            + [pltpu.VMEM((NPT, W), jnp.float32)] * 4
            + [pltpu.VMEM_SHARED((NPAD, W), jnp.float32)]
            + [pltpu.SemaphoreType.DMA] * (2 * _NB)
        ),
    )
    return f(msga, msgb, dst2d, ha, hb, invrep, zeros2d)


def kernel(x, edge_index, edge_attr, params):
    xp = jnp.pad(x, ((0, NPAD - N), (0, 0)))
    eat = jnp.pad(edge_attr, ((0, EPAD - E), (0, 0))).T
    src2d = jnp.pad(edge_index[0], (0, EPAD - E)).reshape(ROWS, EC)
    dst2d = jnp.pad(edge_index[1], (0, EPAD - E),
                    constant_values=NPAD - 1).reshape(ROWS, EC)
    zeros1d = jnp.zeros((NPAD,), jnp.float32)
    zeros2d = jnp.zeros((NPAD, W), jnp.float32)

    cnt = _sc_count(dst2d, zeros1d)
    h0, invrep = _node_enc(xp, cnt.T, params["node_enc"])
    kf0, kf1 = _edge_enc(eat, params["edge_enc"], params["proc"])

    k30, k31 = params["proc"][0]["k3"], params["proc"][1]["k3"]
    hs0a, = _sc_gather([h0], src2d, 0)
    msg0a = _msg(kf0, hs0a, hs0a, k30, fuse=False, half=0)
    hs0b, = _sc_gather([h0], src2d, 1)
    msg0b = _msg(kf0, hs0b, hs0b, k30, fuse=False, half=1)
    q0a, q1a = _sc_scatter(msg0a, msg0b, dst2d, h0, h0, invrep, zeros2d,
                           first=True)

    g1a0, g1a1 = _sc_gather([q0a, q1a], src2d, 0)
    msg1a = _msg(kf1, g1a0, g1a1, k31, fuse=True, half=0)
    g1b0, g1b1 = _sc_gather([q0a, q1a], src2d, 1)
    msg1b = _msg(kf1, g1b0, g1b1, k31, fuse=True, half=1)
    q0b, q1b = _sc_scatter(msg1a, msg1b, dst2d, q0a, q1a, invrep, zeros2d,
                           first=False)

    out = _dec(q0b, q1b, params["dec"])
    return out[:N, :3]
